# baseline (speedup 1.0000x reference)
import functools

import jax
import jax.numpy as jnp
from jax import lax
from jax.experimental import pallas as pl
from jax.experimental.pallas import tpu as pltpu
from jax.experimental.pallas import tpu_sc as plsc

N = 10000
E = 320000
D = 128
H = 128
L = 2
HEADS = 4
T = 2

NC, NS = 2, 16
NW = NC * NS
PER_W = E // NW
CHUNK = 80
G = 5
GC = 25
NBUF = 3
ROWS_PT = 624
TAIL0 = NS * ROWS_PT
TAILR = N - TAIL0

_SQRT_HALF = 0.7071067811865476


def _gelu(v):
    return v * 0.5 * (1.0 + lax.erf(v * _SQRT_HALF))


def _sc_agg(h, em5, zeros):
    mesh = plsc.VectorSubcoreMesh(core_axis_name="c", subcore_axis_name="s")

    @functools.partial(
        pl.kernel,
        out_type=jax.ShapeDtypeStruct((2 * N, H), jnp.float32),
        mesh=mesh,
        scratch_types=[
            [pltpu.VMEM((GC, CHUNK), jnp.int32) for _ in range(2)],
            [pltpu.VMEM((GC, CHUNK), jnp.int32) for _ in range(2)],
            [pltpu.VMEM((CHUNK, H), jnp.float32) for _ in range(NBUF)],
            pltpu.VMEM_SHARED((N, H), jnp.float32),
            [pltpu.SemaphoreType.DMA for _ in range(NBUF)],
            [pltpu.SemaphoreType.DMA for _ in range(2)],
        ],
    )
    def body(e_hbm, h_hbm, z_hbm, out_hbm,
             sidx, didx, rows, acc, gsems, isems):
        cid = lax.axis_index("c")
        sid = lax.axis_index("s")
        wid = sid * NC + cid

        def start_idx(g, p):
            pltpu.async_copy(e_hbm.at[0, wid, g], sidx[p], isems[p])
            pltpu.async_copy(e_hbm.at[1, wid, g], didx[p], isems[p])

        def wait_idx(g, p):
            pltpu.make_async_copy(e_hbm.at[0, wid, g], sidx[p], isems[p]).wait()
            pltpu.make_async_copy(e_hbm.at[1, wid, g], didx[p], isems[p]).wait()

        start_idx(0, 0)
        r0 = sid * ROWS_PT
        pltpu.sync_copy(z_hbm.at[pl.ds(r0, ROWS_PT)], acc.at[pl.ds(r0, ROWS_PT)])

        @pl.when(sid == NS - 1)
        def _zero_tail():
            pltpu.sync_copy(z_hbm.at[pl.ds(TAIL0, TAILR)],
                            acc.at[pl.ds(TAIL0, TAILR)])

        plsc.subcore_barrier()

        for g in range(G):
            p = g % 2
            wait_idx(g, p)
            if g + 1 < G:
                start_idx(g + 1, (g + 1) % 2)

            def start_g(j, b, p=p):
                pltpu.async_copy(h_hbm.at[sidx[p].at[j]], rows[b], gsems[b])

            def wait_g(j, b, p=p):
                pltpu.make_async_copy(h_hbm.at[sidx[p].at[j]], rows[b],
                                      gsems[b]).wait()

            def scat(j, b, p=p):
                pltpu.sync_copy(rows[b], acc.at[didx[p].at[j]], add=True)

            for b in range(NBUF):
                start_g(b, b)

            def duo(i, carry):
                j0 = i * NBUF
                for b in range(NBUF):
                    j = j0 + b
                    wait_g(j, b)
                    scat(j, b)

                    @pl.when(j + NBUF < GC)
                    def _launch(j=j, b=b):
                        start_g(j + NBUF, b)
                return carry

            lax.fori_loop(0, GC // NBUF, duo, 0)
            wait_g(GC - 1, (GC - 1) % NBUF)
            scat(GC - 1, (GC - 1) % NBUF)

        plsc.subcore_barrier()
        pltpu.sync_copy(acc.at[pl.ds(r0, ROWS_PT)],
                        out_hbm.at[pl.ds(cid * N + r0, ROWS_PT)])

        @pl.when(sid == NS - 1)
        def _flush_tail():
            pltpu.sync_copy(acc.at[pl.ds(TAIL0, TAILR)],
                            out_hbm.at[pl.ds(cid * N + TAIL0, TAILR)])

    return body(em5, h, zeros)


BN = 2000


def _enc_body(x_ref, w_ref, b_ref, o_ref):
    o_ref[...] = _gelu(
        jnp.dot(x_ref[...], w_ref[...], preferred_element_type=jnp.float32)
        + b_ref[...])


def _encoder(x, w, b):
    return pl.pallas_call(
        _enc_body,
        grid=(N // BN,),
        in_specs=[
            pl.BlockSpec((BN, D), lambda i: (i, 0)),
            pl.BlockSpec((D, H), lambda i: (0, 0)),
            pl.BlockSpec((1, H), lambda i: (0, 0)),
        ],
        out_specs=pl.BlockSpec((BN, H), lambda i: (i, 0)),
        out_shape=jax.ShapeDtypeStruct((N, H), jnp.float32),
    )(x, w, b)


def _root_body(h_ref, w_ref, o_ref):
    o_ref[...] = jnp.dot(h_ref[...], w_ref[...],
                         preferred_element_type=jnp.float32)


def _root(h, w):
    return pl.pallas_call(
        _root_body,
        grid=(N // BN,),
        in_specs=[
            pl.BlockSpec((BN, H), lambda i: (i, 0)),
            pl.BlockSpec((H, H), lambda i: (0, 0)),
        ],
        out_specs=pl.BlockSpec((BN, H), lambda i: (i, 0)),
        out_shape=jax.ShapeDtypeStruct((N, H), jnp.float32),
    )(h, w)


def _blk_body(p0_ref, p1_ref, root_ref, h_ref, wrel_ref, brel_ref,
              g_ref, b_ref, o_ref):
    agg = p0_ref[...] + p1_ref[...]
    hv = h_ref[...]
    conv = (jnp.dot(agg, wrel_ref[...], preferred_element_type=jnp.float32)
            + brel_ref[...] + root_ref[...])
    hh = _gelu(conv) + hv
    mu = jnp.mean(hh, axis=-1, keepdims=True)
    var = jnp.mean((hh - mu) ** 2, axis=-1, keepdims=True)
    o_ref[...] = (hh - mu) * lax.rsqrt(var + 1e-5) * g_ref[...] + b_ref[...]


def _block(partials, root, h, wrel, brel, g, b):
    nb = N // BN
    return pl.pallas_call(
        _blk_body,
        grid=(nb,),
        in_specs=[
            pl.BlockSpec((BN, H), lambda i: (i, 0)),
            pl.BlockSpec((BN, H), lambda i, nb=nb: (i + nb, 0)),
            pl.BlockSpec((BN, H), lambda i: (i, 0)),
            pl.BlockSpec((BN, H), lambda i: (i, 0)),
            pl.BlockSpec((H, H), lambda i: (0, 0)),
            pl.BlockSpec((1, H), lambda i: (0, 0)),
            pl.BlockSpec((1, H), lambda i: (0, 0)),
            pl.BlockSpec((1, H), lambda i: (0, 0)),
        ],
        out_specs=pl.BlockSpec((BN, H), lambda i: (i, 0)),
        out_shape=jax.ShapeDtypeStruct((N, H), jnp.float32),
    )(partials, partials, root, h, wrel, brel, g, b)


def _dec_body(h_ref, gw_ref, gb_ref, w1_ref, b1_ref, w2_ref, b2_ref, o_ref):
    hv = h_ref[...]
    g = jnp.dot(hv, gw_ref[...], preferred_element_type=jnp.float32) + gb_ref[...]
    e = _gelu(jnp.dot(hv, w1_ref[...], preferred_element_type=jnp.float32)
              + b1_ref[...])
    ho = jnp.dot(e, w2_ref[...], preferred_element_type=jnp.float32) + b2_ref[...]
    cols = []
    for t in range(T):
        gt = g[:, t * HEADS:(t + 1) * HEADS]
        m = jnp.max(gt, axis=-1, keepdims=True)
        ex = jnp.exp(gt - m)
        wsm = ex / jnp.sum(ex, axis=-1, keepdims=True)
        cols.append(jnp.sum(wsm * ho[:, t * HEADS:(t + 1) * HEADS],
                            axis=-1, keepdims=True))
    o_ref[...] = jnp.concatenate(cols, axis=-1)


DBN = 2000


def _decoder(h, gw, gb, w1, b1, w2, b2):
    nh = T * HEADS
    return pl.pallas_call(
        _dec_body,
        grid=(N // DBN,),
        in_specs=[
            pl.BlockSpec((DBN, H), lambda i: (i, 0)),
            pl.BlockSpec((H, nh), lambda i: (0, 0)),
            pl.BlockSpec((1, nh), lambda i: (0, 0)),
            pl.BlockSpec((H, nh * H), lambda i: (0, 0)),
            pl.BlockSpec((1, nh * H), lambda i: (0, 0)),
            pl.BlockSpec((nh * H, nh), lambda i: (0, 0)),
            pl.BlockSpec((1, nh), lambda i: (0, 0)),
        ],
        out_specs=pl.BlockSpec((DBN, T), lambda i: (i, 0)),
        out_shape=jax.ShapeDtypeStruct((N, T), jnp.float32),
    )(h, gw, gb, w1, b1, w2, b2)


def kernel(x, edge_index, params):
    em5 = edge_index.reshape(2, NW, G, GC, CHUNK)
    zeros = jnp.zeros((N, H), jnp.float32)

    h = _encoder(x, params['enc_w'], params['enc_b'].reshape(1, H))
    for l in range(L):
        partials = _sc_agg(h, em5, zeros)
        root = _root(h, params[f'blk{l}_wroot'])
        h = _block(partials, root, h,
                   params[f'blk{l}_wrel'], params[f'blk{l}_brel'].reshape(1, H),
                   params[f'blk{l}_ln_g'].reshape(1, H),
                   params[f'blk{l}_ln_b'].reshape(1, H))

    nh = T * HEADS
    gw = jnp.concatenate([params[f'dec{t}_gate_w'] for t in range(T)], axis=1)
    gb = jnp.concatenate([params[f'dec{t}_gate_b'] for t in range(T)]).reshape(1, nh)
    w1 = jnp.concatenate(
        [params[f'dec{t}_h{k}_w1'] for t in range(T) for k in range(HEADS)], axis=1)
    b1 = jnp.concatenate(
        [params[f'dec{t}_h{k}_b1'] for t in range(T) for k in range(HEADS)]
    ).reshape(1, nh * H)
    w2cols = [params[f'dec{t}_h{k}_w2'][:, 0] for t in range(T) for k in range(HEADS)]
    w2 = jnp.zeros((nh * H, nh), jnp.float32)
    for j, col in enumerate(w2cols):
        w2 = w2.at[j * H:(j + 1) * H, j].set(col)
    b2 = jnp.stack(
        [params[f'dec{t}_h{k}_b2'][0] for t in range(T) for k in range(HEADS)]
    ).reshape(1, nh)

    return _decoder(h, gw, gb, w1, b1, w2, b2)


# speedup vs baseline: 1.0292x; 1.0292x over previous
import functools

import jax
import jax.numpy as jnp
from jax import lax
from jax.experimental import pallas as pl
from jax.experimental.pallas import tpu as pltpu
from jax.experimental.pallas import tpu_sc as plsc

N = 10000
E = 320000
D = 128
H = 128
L = 2
HEADS = 4
T = 2

NC, NS = 2, 16
NW = NC * NS
PER_W = E // NW
CHUNK = 80
G = 5
GC = 25
NBUF = 3
ROWS_PT = 624
TAIL0 = NS * ROWS_PT
TAILR = N - TAIL0

_SQRT_HALF = 0.7071067811865476


def _gelu(v):
    return v * 0.5 * (1.0 + lax.erf(v * _SQRT_HALF))


def _sc_agg(h, em5, zeros):
    mesh = plsc.VectorSubcoreMesh(core_axis_name="c", subcore_axis_name="s")

    @functools.partial(
        pl.kernel,
        out_type=jax.ShapeDtypeStruct((2 * N, H), jnp.float32),
        mesh=mesh,
        scratch_types=[
            [pltpu.VMEM((GC, CHUNK), jnp.int32) for _ in range(2)],
            [pltpu.VMEM((GC, CHUNK), jnp.int32) for _ in range(2)],
            [pltpu.VMEM((CHUNK, H), jnp.float32) for _ in range(NBUF)],
            pltpu.VMEM_SHARED((N, H), jnp.float32),
            [pltpu.SemaphoreType.DMA for _ in range(NBUF)],
            [pltpu.SemaphoreType.DMA for _ in range(2)],
        ],
    )
    def body(e_hbm, h_hbm, z_hbm, out_hbm,
             sidx, didx, rows, acc, gsems, isems):
        cid = lax.axis_index("c")
        sid = lax.axis_index("s")
        wid = sid * NC + cid

        def start_idx(g, p):
            pltpu.async_copy(e_hbm.at[0, wid, g], sidx[p], isems[p])
            pltpu.async_copy(e_hbm.at[1, wid, g], didx[p], isems[p])

        def wait_idx(g, p):
            pltpu.make_async_copy(e_hbm.at[0, wid, g], sidx[p], isems[p]).wait()
            pltpu.make_async_copy(e_hbm.at[1, wid, g], didx[p], isems[p]).wait()

        start_idx(0, 0)
        r0 = sid * ROWS_PT
        pltpu.sync_copy(z_hbm.at[pl.ds(r0, ROWS_PT)], acc.at[pl.ds(r0, ROWS_PT)])

        @pl.when(sid == NS - 1)
        def _zero_tail():
            pltpu.sync_copy(z_hbm.at[pl.ds(TAIL0, TAILR)],
                            acc.at[pl.ds(TAIL0, TAILR)])

        plsc.subcore_barrier()

        for g in range(G):
            p = g % 2
            wait_idx(g, p)
            if g + 1 < G:
                start_idx(g + 1, (g + 1) % 2)

            def start_g(j, b, p=p):
                pltpu.async_copy(h_hbm.at[sidx[p].at[j]], rows[b], gsems[b])

            def wait_g(j, b, p=p):
                pltpu.make_async_copy(h_hbm.at[sidx[p].at[j]], rows[b],
                                      gsems[b]).wait()

            def scat(j, b, p=p):
                pltpu.sync_copy(rows[b], acc.at[didx[p].at[j]], add=True)

            for b in range(NBUF):
                start_g(b, b)

            def duo(i, carry):
                j0 = i * NBUF
                for b in range(NBUF):
                    j = j0 + b
                    wait_g(j, b)
                    scat(j, b)

                    @pl.when(j + NBUF < GC)
                    def _launch(j=j, b=b):
                        start_g(j + NBUF, b)
                return carry

            lax.fori_loop(0, GC // NBUF, duo, 0)
            wait_g(GC - 1, (GC - 1) % NBUF)
            scat(GC - 1, (GC - 1) % NBUF)

        plsc.subcore_barrier()
        pltpu.sync_copy(acc.at[pl.ds(r0, ROWS_PT)],
                        out_hbm.at[pl.ds(cid * N + r0, ROWS_PT)])

        @pl.when(sid == NS - 1)
        def _flush_tail():
            pltpu.sync_copy(acc.at[pl.ds(TAIL0, TAILR)],
                            out_hbm.at[pl.ds(cid * N + TAIL0, TAILR)])

    return body(em5, h, zeros)


BN = 2000


def _enc_body(x_ref, w_ref, b_ref, o_ref):
    o_ref[...] = _gelu(
        jnp.dot(x_ref[...], w_ref[...], preferred_element_type=jnp.float32)
        + b_ref[...])


def _encoder(x, w, b):
    return pl.pallas_call(
        _enc_body,
        grid=(N // BN,),
        in_specs=[
            pl.BlockSpec((BN, D), lambda i: (i, 0)),
            pl.BlockSpec((D, H), lambda i: (0, 0)),
            pl.BlockSpec((1, H), lambda i: (0, 0)),
        ],
        out_specs=pl.BlockSpec((BN, H), lambda i: (i, 0)),
        out_shape=jax.ShapeDtypeStruct((N, H), jnp.float32),
    )(x, w, b)


def _blk_body(p0_ref, p1_ref, h_ref, wrel_ref, brel_ref, wroot_ref,
              g_ref, b_ref, o_ref):
    agg = p0_ref[...] + p1_ref[...]
    hv = h_ref[...]
    conv = (jnp.dot(agg, wrel_ref[...], preferred_element_type=jnp.float32)
            + brel_ref[...]
            + jnp.dot(hv, wroot_ref[...], preferred_element_type=jnp.float32))
    hh = _gelu(conv) + hv
    mu = jnp.mean(hh, axis=-1, keepdims=True)
    var = jnp.mean((hh - mu) ** 2, axis=-1, keepdims=True)
    o_ref[...] = (hh - mu) * lax.rsqrt(var + 1e-5) * g_ref[...] + b_ref[...]


def _block(partials, h, wrel, brel, wroot, g, b):
    nb = N // BN
    return pl.pallas_call(
        _blk_body,
        grid=(nb,),
        in_specs=[
            pl.BlockSpec((BN, H), lambda i: (i, 0)),
            pl.BlockSpec((BN, H), lambda i, nb=nb: (i + nb, 0)),
            pl.BlockSpec((BN, H), lambda i: (i, 0)),
            pl.BlockSpec((H, H), lambda i: (0, 0)),
            pl.BlockSpec((1, H), lambda i: (0, 0)),
            pl.BlockSpec((H, H), lambda i: (0, 0)),
            pl.BlockSpec((1, H), lambda i: (0, 0)),
            pl.BlockSpec((1, H), lambda i: (0, 0)),
        ],
        out_specs=pl.BlockSpec((BN, H), lambda i: (i, 0)),
        out_shape=jax.ShapeDtypeStruct((N, H), jnp.float32),
    )(partials, partials, h, wrel, brel, wroot, g, b)


def _dec_math(hv, gw_ref, gb_ref, w1_ref, b1_ref, w2_ref, b2_ref):
    g = jnp.dot(hv, gw_ref[...], preferred_element_type=jnp.float32) + gb_ref[...]
    e = _gelu(jnp.dot(hv, w1_ref[...], preferred_element_type=jnp.float32)
              + b1_ref[...])
    ho = jnp.dot(e, w2_ref[...], preferred_element_type=jnp.float32) + b2_ref[...]
    cols = []
    for t in range(T):
        gt = g[:, t * HEADS:(t + 1) * HEADS]
        m = jnp.max(gt, axis=-1, keepdims=True)
        ex = jnp.exp(gt - m)
        wsm = ex / jnp.sum(ex, axis=-1, keepdims=True)
        cols.append(jnp.sum(wsm * ho[:, t * HEADS:(t + 1) * HEADS],
                            axis=-1, keepdims=True))
    return jnp.concatenate(cols, axis=-1)


def _blkdec_body(p0_ref, p1_ref, h_ref, wrel_ref, brel_ref, wroot_ref,
                 g_ref, b_ref, gw_ref, gb_ref, w1_ref, b1_ref, w2_ref,
                 b2_ref, o_ref):
    agg = p0_ref[...] + p1_ref[...]
    hv = h_ref[...]
    conv = (jnp.dot(agg, wrel_ref[...], preferred_element_type=jnp.float32)
            + brel_ref[...]
            + jnp.dot(hv, wroot_ref[...], preferred_element_type=jnp.float32))
    hh = _gelu(conv) + hv
    mu = jnp.mean(hh, axis=-1, keepdims=True)
    var = jnp.mean((hh - mu) ** 2, axis=-1, keepdims=True)
    hln = (hh - mu) * lax.rsqrt(var + 1e-5) * g_ref[...] + b_ref[...]
    o_ref[...] = _dec_math(hln, gw_ref, gb_ref, w1_ref, b1_ref,
                           w2_ref, b2_ref)


def _blkdec(partials, h, wrel, brel, wroot, g, b, gw, gb, w1, b1, w2, b2):
    nb = N // BN
    nh = T * HEADS
    return pl.pallas_call(
        _blkdec_body,
        grid=(nb,),
        in_specs=[
            pl.BlockSpec((BN, H), lambda i: (i, 0)),
            pl.BlockSpec((BN, H), lambda i, nb=nb: (i + nb, 0)),
            pl.BlockSpec((BN, H), lambda i: (i, 0)),
            pl.BlockSpec((H, H), lambda i: (0, 0)),
            pl.BlockSpec((1, H), lambda i: (0, 0)),
            pl.BlockSpec((H, H), lambda i: (0, 0)),
            pl.BlockSpec((1, H), lambda i: (0, 0)),
            pl.BlockSpec((1, H), lambda i: (0, 0)),
            pl.BlockSpec((H, nh), lambda i: (0, 0)),
            pl.BlockSpec((1, nh), lambda i: (0, 0)),
            pl.BlockSpec((H, nh * H), lambda i: (0, 0)),
            pl.BlockSpec((1, nh * H), lambda i: (0, 0)),
            pl.BlockSpec((nh * H, nh), lambda i: (0, 0)),
            pl.BlockSpec((1, nh), lambda i: (0, 0)),
        ],
        out_specs=pl.BlockSpec((BN, T), lambda i: (i, 0)),
        out_shape=jax.ShapeDtypeStruct((N, T), jnp.float32),
    )(partials, partials, h, wrel, brel, wroot, g, b, gw, gb, w1, b1, w2, b2)


def kernel(x, edge_index, params):
    em5 = edge_index.reshape(2, NW, G, GC, CHUNK)
    zeros = jnp.zeros((N, H), jnp.float32)

    h = _encoder(x, params['enc_w'], params['enc_b'].reshape(1, H))
    partials = _sc_agg(h, em5, zeros)
    h = _block(partials, h,
               params['blk0_wrel'], params['blk0_brel'].reshape(1, H),
               params['blk0_wroot'],
               params['blk0_ln_g'].reshape(1, H),
               params['blk0_ln_b'].reshape(1, H))
    partials = _sc_agg(h, em5, zeros)

    nh = T * HEADS
    gw = jnp.concatenate([params[f'dec{t}_gate_w'] for t in range(T)], axis=1)
    gb = jnp.concatenate([params[f'dec{t}_gate_b'] for t in range(T)]).reshape(1, nh)
    w1 = jnp.concatenate(
        [params[f'dec{t}_h{k}_w1'] for t in range(T) for k in range(HEADS)], axis=1)
    b1 = jnp.concatenate(
        [params[f'dec{t}_h{k}_b1'] for t in range(T) for k in range(HEADS)]
    ).reshape(1, nh * H)
    w2cols = [params[f'dec{t}_h{k}_w2'][:, 0] for t in range(T) for k in range(HEADS)]
    w2 = jnp.zeros((nh * H, nh), jnp.float32)
    for j, col in enumerate(w2cols):
        w2 = w2.at[j * H:(j + 1) * H, j].set(col)
    b2 = jnp.stack(
        [params[f'dec{t}_h{k}_b2'][0] for t in range(T) for k in range(HEADS)]
    ).reshape(1, nh)

    return _blkdec(partials, h,
                   params['blk1_wrel'], params['blk1_brel'].reshape(1, H),
                   params['blk1_wroot'],
                   params['blk1_ln_g'].reshape(1, H),
                   params['blk1_ln_b'].reshape(1, H),
                   gw, gb, w1, b1, w2, b2)
